# baseline (speedup 1.0000x reference)
import functools

import jax
import jax.numpy as jnp
from jax import lax
from jax.experimental import pallas as pl
from jax.experimental.pallas import tpu as pltpu
from jax.experimental.pallas import tpu_sc as plsc

N = 10000
E = 320000
D = 128
NCLS = 40

NC = 2
NS = 16
NW = NC * NS
EPW = E // NW
EB = 128
NFULL = EPW // EB
EREM = EPW - NFULL * EB
NB = E // EB
NBW = NB // NW
NBX = NB - NBW * NW
ARS = 624
ACH = 104
NCH = ARS // ACH
TAIL = N - ARS * NS
DZ = 624

RB = 1000
G = N // RB

@functools.lru_cache(maxsize=None)
def _sc_kernels():
    mesh = plsc.VectorSubcoreMesh(core_axis_name="c", subcore_axis_name="s",
                                  num_cores=NC, num_subcores=NS)
    deg = _make_sc_deg(mesh)
    agg = _make_sc_agg(mesh)
    return deg, agg


def _make_sc_deg(mesh):
    return functools.partial(
        pl.kernel,
        out_type=jax.ShapeDtypeStruct((NC * N,), jnp.float32),
        mesh=mesh,
        scratch_types=[
            pltpu.VMEM((NBW * EB,), jnp.int32),
            pltpu.VMEM((NBX * EB,), jnp.int32),
            pltpu.VMEM((NBW * EB,), jnp.float32),
            pltpu.VMEM((DZ + 16,), jnp.float32),
            pltpu.VMEM_SHARED((N,), jnp.float32),
            pltpu.SemaphoreType.DMA,
        ],
    )(_sc_deg_body)


def _sc_deg_body(dst_hbm, out_hbm, dstv, dstx, ones, zb, acc, sem):
    c = lax.axis_index("c")
    s = lax.axis_index("s")
    w = c * NS + s

    base = w * NBW * EB
    pltpu.async_copy(dst_hbm.at[pl.ds(base, NBW * EB)], dstv, sem)

    @pl.when(w == NW - 1)
    def _():
        pltpu.sync_copy(dst_hbm.at[pl.ds(NBW * NW * EB, NBX * EB)], dstx)

    def fill_z(i, carry):
        zb[pl.ds(i * 16, 16)] = jnp.zeros((16,), jnp.float32)
        return carry

    lax.fori_loop(0, (DZ + 16) // 16, fill_z, 0)

    def fill_o(i, carry):
        ones[pl.ds(i * 16, 16)] = jnp.ones((16,), jnp.float32)
        return carry

    lax.fori_loop(0, NBW * EB // 16, fill_o, 0)

    pltpu.sync_copy(zb.at[pl.ds(0, DZ)], acc.at[pl.ds(s * DZ, DZ)])

    @pl.when(s == NS - 1)
    def _():
        pltpu.sync_copy(zb.at[pl.ds(0, 16)], acc.at[pl.ds(DZ * NS, 16)])

    plsc.subcore_barrier()
    pltpu.make_async_copy(dst_hbm.at[pl.ds(base, NBW * EB)], dstv, sem).wait()

    pltpu.sync_copy(ones, acc.at[dstv], add=True)

    @pl.when(w == NW - 1)
    def _():
        pltpu.sync_copy(ones.at[pl.ds(0, NBX * EB)], acc.at[dstx], add=True)

    plsc.subcore_barrier()

    pltpu.sync_copy(acc.at[pl.ds(s * DZ, DZ)], zb.at[pl.ds(0, DZ)])
    pltpu.sync_copy(zb.at[pl.ds(0, DZ)], out_hbm.at[pl.ds(c * N + s * DZ, DZ)])

    @pl.when(s == NS - 1)
    def _():
        pltpu.sync_copy(acc.at[pl.ds(DZ * NS, 16)], zb.at[pl.ds(0, 16)])
        pltpu.sync_copy(zb.at[pl.ds(0, 16)],
                        out_hbm.at[pl.ds(c * N + DZ * NS, 16)])


EPB = 192
NBA = (E // NW) // EPB
EXX = E - NW * NBA * EPB
XEB = 128


def _make_sc_agg(mesh):
    return functools.partial(
        pl.kernel,
        out_type=jax.ShapeDtypeStruct((NC, N, D), jnp.float32),
        mesh=mesh,
        scratch_types=[
            pltpu.VMEM((EPB,), jnp.int32),
            pltpu.VMEM((EPB,), jnp.int32),
            pltpu.VMEM((EPB,), jnp.int32),
            pltpu.VMEM((EPB,), jnp.int32),
            pltpu.VMEM((XEB,), jnp.int32),
            pltpu.VMEM((EPB, D), jnp.float32),
            pltpu.VMEM((EPB, D), jnp.float32),
            pltpu.VMEM_SHARED((N, D), jnp.float32),
            pltpu.SemaphoreType.DMA,
            pltpu.SemaphoreType.DMA,
        ],
    )(_sc_agg_body)


def _sc_agg_body(y_hbm, src_hbm, dst_hbm, out_hbm, srcb0, srcb1, dstb0,
                 dstb1, dstx, rows0, rows1, acc, gsem0, gsem1):
    c = lax.axis_index("c")
    s = lax.axis_index("s")
    w = c * NS + s

    eb0 = w * NBA * EPB

    pltpu.sync_copy(src_hbm.at[pl.ds(eb0, EPB)], srcb0)
    pltpu.sync_copy(dst_hbm.at[pl.ds(eb0, EPB)], dstb0)
    pltpu.sync_copy(src_hbm.at[pl.ds(eb0 + EPB, EPB)], srcb1)
    pltpu.sync_copy(dst_hbm.at[pl.ds(eb0 + EPB, EPB)], dstb1)
    pltpu.async_copy(y_hbm.at[srcb0], rows1, gsem1)

    def zrow(i, carry):
        def zcol(j, carry2):
            rows0[i, pl.ds(j * 16, 16)] = jnp.zeros((16,), jnp.float32)
            return carry2

        lax.fori_loop(0, D // 16, zcol, 0)
        return carry

    lax.fori_loop(0, EPB, zrow, 0)

    for k in range(NCH):
        pltpu.sync_copy(rows0.at[pl.ds(0, ACH)],
                        acc.at[pl.ds(s * ARS + k * ACH, ACH)])

    @pl.when(s == NS - 1)
    def _():
        pltpu.sync_copy(rows0.at[pl.ds(0, TAIL)], acc.at[pl.ds(ARS * NS, TAIL)])

    pltpu.async_copy(y_hbm.at[srcb1], rows0, gsem0)

    plsc.subcore_barrier()

    def pair(k, carry):
        g0 = 2 * k
        pltpu.make_async_copy(y_hbm.at[srcb0], rows1, gsem1).wait()
        pltpu.sync_copy(rows1, acc.at[dstb0], add=True)

        @pl.when(g0 + 2 < NBA)
        def _():
            pltpu.sync_copy(src_hbm.at[pl.ds(eb0 + (g0 + 2) * EPB, EPB)],
                            srcb0)
            pltpu.sync_copy(dst_hbm.at[pl.ds(eb0 + (g0 + 2) * EPB, EPB)],
                            dstb0)
            pltpu.async_copy(y_hbm.at[srcb0], rows1, gsem1)

        pltpu.make_async_copy(y_hbm.at[srcb1], rows0, gsem0).wait()
        pltpu.sync_copy(rows0, acc.at[dstb1], add=True)

        @pl.when(g0 + 3 < NBA)
        def _():
            pltpu.sync_copy(src_hbm.at[pl.ds(eb0 + (g0 + 3) * EPB, EPB)],
                            srcb1)
            pltpu.sync_copy(dst_hbm.at[pl.ds(eb0 + (g0 + 3) * EPB, EPB)],
                            dstb1)
            pltpu.async_copy(y_hbm.at[srcb1], rows0, gsem0)

        return carry

    lax.fori_loop(0, NBA // 2, pair, 0)

    @pl.when(w == NW - 1)
    def _():
        def extra(i, carry):
            e0 = NW * NBA * EPB + i * XEB
            pltpu.sync_copy(src_hbm.at[pl.ds(e0, XEB)], srcb0.at[pl.ds(0, XEB)])
            pltpu.sync_copy(dst_hbm.at[pl.ds(e0, XEB)], dstx)
            pltpu.async_copy(y_hbm.at[srcb0.at[pl.ds(0, XEB)]],
                             rows0.at[pl.ds(0, XEB)], gsem0).wait()
            pltpu.sync_copy(rows0.at[pl.ds(0, XEB)], acc.at[dstx], add=True)
            return carry

        lax.fori_loop(0, EXX // XEB, extra, 0)

    plsc.subcore_barrier()

    for k in range(NCH):
        r0 = s * ARS + k * ACH
        pltpu.sync_copy(acc.at[pl.ds(r0, ACH)], rows0.at[pl.ds(0, ACH)])
        pltpu.sync_copy(rows0.at[pl.ds(0, ACH)], out_hbm.at[c, pl.ds(r0, ACH)])

    @pl.when(s == NS - 1)
    def _():
        r0 = ARS * NS
        pltpu.sync_copy(acc.at[pl.ds(r0, TAIL)], rows1.at[pl.ds(0, TAIL)])
        pltpu.sync_copy(rows1.at[pl.ds(0, TAIL)], out_hbm.at[c, pl.ds(r0, TAIL)])


def _tc1_body(degp_ref, x_ref, w_ref, y_ref, dinv_ref):
    deg = degp_ref[0] + degp_ref[1] + 1.0
    dinv = lax.rsqrt(jnp.maximum(deg, 1e-12))
    xw = jnp.dot(x_ref[...], w_ref[...], preferred_element_type=jnp.float32)
    y_ref[...] = xw * dinv
    dinv_ref[...] = dinv


def _tc1(degp, x, W1):
    return pl.pallas_call(
        _tc1_body,
        grid=(G,),
        in_specs=[
            pl.BlockSpec((NC, RB, 1), lambda i: (0, i, 0)),
            pl.BlockSpec((RB, D), lambda i: (i, 0)),
            pl.BlockSpec((D, D), lambda i: (0, 0)),
        ],
        out_specs=[
            pl.BlockSpec((RB, D), lambda i: (i, 0)),
            pl.BlockSpec((RB, 1), lambda i: (i, 0)),
        ],
        out_shape=[
            jax.ShapeDtypeStruct((N, D), jnp.float32),
            jax.ShapeDtypeStruct((N, 1), jnp.float32),
        ],
    )(degp, x, W1)


def _tc_mid_body(a_ref, y_ref, dinv_ref, b_ref, w_ref, yn_ref):
    dinv = dinv_ref[...]
    h = dinv * (a_ref[0] + a_ref[1] + y_ref[...]) + b_ref[...]
    h = jnp.maximum(h, 0.0)
    yn_ref[...] = jnp.dot(h, w_ref[...],
                          preferred_element_type=jnp.float32) * dinv


def _tc_mid(a, y, dinv, b, W):
    return pl.pallas_call(
        _tc_mid_body,
        grid=(G,),
        in_specs=[
            pl.BlockSpec((NC, RB, D), lambda i: (0, i, 0)),
            pl.BlockSpec((RB, D), lambda i: (i, 0)),
            pl.BlockSpec((RB, 1), lambda i: (i, 0)),
            pl.BlockSpec((1, D), lambda i: (0, 0)),
            pl.BlockSpec((D, D), lambda i: (0, 0)),
        ],
        out_specs=pl.BlockSpec((RB, D), lambda i: (i, 0)),
        out_shape=jax.ShapeDtypeStruct((N, D), jnp.float32),
    )(a, y, dinv, b, W)


def _tc_final_body(a_ref, y_ref, dinv_ref, b3_ref, wf1_ref, bf1_ref, wf2_ref,
                   bf2_ref, out_ref):
    h = dinv_ref[...] * (a_ref[0] + a_ref[1] + y_ref[...]) + b3_ref[...]
    h = jnp.maximum(h, 0.0)
    z = jnp.dot(h, wf1_ref[...],
                preferred_element_type=jnp.float32) + bf1_ref[...]
    z = jnp.maximum(z, 0.0)
    out_ref[...] = jnp.dot(z, wf2_ref[...],
                           preferred_element_type=jnp.float32) + bf2_ref[...]


def _tc_final(a, y, dinv, b3, Wf1, bf1, Wf2, bf2):
    return pl.pallas_call(
        _tc_final_body,
        grid=(G,),
        in_specs=[
            pl.BlockSpec((NC, RB, D), lambda i: (0, i, 0)),
            pl.BlockSpec((RB, D), lambda i: (i, 0)),
            pl.BlockSpec((RB, 1), lambda i: (i, 0)),
            pl.BlockSpec((1, D), lambda i: (0, 0)),
            pl.BlockSpec((D, D), lambda i: (0, 0)),
            pl.BlockSpec((1, D), lambda i: (0, 0)),
            pl.BlockSpec((D, NCLS), lambda i: (0, 0)),
            pl.BlockSpec((1, NCLS), lambda i: (0, 0)),
        ],
        out_specs=pl.BlockSpec((RB, NCLS), lambda i: (i, 0)),
        out_shape=jax.ShapeDtypeStruct((N, NCLS), jnp.float32),
    )(a, y, dinv, b3, Wf1, bf1, Wf2, bf2)


def kernel(x, edge_index, batch, W1, b1, W2, b2, W3, b3, Wf1, bf1, Wf2, bf2):
    src = edge_index[0]
    dst = edge_index[1]

    _sc_deg, _sc_agg = _sc_kernels()
    degp = _sc_deg(dst)
    y1, dinv = _tc1(degp.reshape(NC, N, 1), x, W1)
    a1 = _sc_agg(y1, src, dst)
    y2 = _tc_mid(a1, y1, dinv, b1.reshape(1, D), W2)
    a2 = _sc_agg(y2, src, dst)
    y3 = _tc_mid(a2, y2, dinv, b2.reshape(1, D), W3)
    a3 = _sc_agg(y3, src, dst)
    out = _tc_final(a3, y3, dinv, b3.reshape(1, D), Wf1, bf1.reshape(1, D),
                    Wf2, bf2.reshape(1, NCLS))
    return out


# speedup vs baseline: 1.0572x; 1.0572x over previous
import functools

import jax
import jax.numpy as jnp
from jax import lax
from jax.experimental import pallas as pl
from jax.experimental.pallas import tpu as pltpu
from jax.experimental.pallas import tpu_sc as plsc

N = 10000
E = 320000
D = 128
NCLS = 40

NC = 2
NS = 16
NW = NC * NS
EPW = E // NW
EB = 128
NFULL = EPW // EB
EREM = EPW - NFULL * EB
NB = E // EB
NBW = NB // NW
NBX = NB - NBW * NW
ARS = 624
ACH = 104
NCH = ARS // ACH
TAIL = N - ARS * NS
DZ = 624

RB = 1000
G = N // RB

@functools.lru_cache(maxsize=None)
def _sc_kernels():
    mesh = plsc.VectorSubcoreMesh(core_axis_name="c", subcore_axis_name="s",
                                  num_cores=NC, num_subcores=NS)
    deg = _make_sc_deg(mesh)
    agg = _make_sc_agg(mesh)
    return deg, agg


def _make_sc_deg(mesh):
    return functools.partial(
        pl.kernel,
        out_type=jax.ShapeDtypeStruct((NC * N,), jnp.float32),
        mesh=mesh,
        scratch_types=[
            pltpu.VMEM((NBW * EB,), jnp.int32),
            pltpu.VMEM((NBX * EB,), jnp.int32),
            pltpu.VMEM((NBW * EB,), jnp.float32),
            pltpu.VMEM((DZ + 16,), jnp.float32),
            pltpu.VMEM_SHARED((N,), jnp.float32),
            pltpu.SemaphoreType.DMA,
        ],
    )(_sc_deg_body)


def _sc_deg_body(dst_hbm, out_hbm, dstv, dstx, ones, zb, acc, sem):
    c = lax.axis_index("c")
    s = lax.axis_index("s")
    w = c * NS + s

    base = w * NBW * EB
    pltpu.async_copy(dst_hbm.at[pl.ds(base, NBW * EB)], dstv, sem)

    @pl.when(w == NW - 1)
    def _():
        pltpu.sync_copy(dst_hbm.at[pl.ds(NBW * NW * EB, NBX * EB)], dstx)

    def fill_z(i, carry):
        zb[pl.ds(i * 16, 16)] = jnp.zeros((16,), jnp.float32)
        return carry

    lax.fori_loop(0, (DZ + 16) // 16, fill_z, 0)

    def fill_o(i, carry):
        ones[pl.ds(i * 16, 16)] = jnp.ones((16,), jnp.float32)
        return carry

    lax.fori_loop(0, NBW * EB // 16, fill_o, 0)

    pltpu.sync_copy(zb.at[pl.ds(0, DZ)], acc.at[pl.ds(s * DZ, DZ)])

    @pl.when(s == NS - 1)
    def _():
        pltpu.sync_copy(zb.at[pl.ds(0, 16)], acc.at[pl.ds(DZ * NS, 16)])

    plsc.subcore_barrier()
    pltpu.make_async_copy(dst_hbm.at[pl.ds(base, NBW * EB)], dstv, sem).wait()

    pltpu.sync_copy(ones, acc.at[dstv], add=True)

    @pl.when(w == NW - 1)
    def _():
        pltpu.sync_copy(ones.at[pl.ds(0, NBX * EB)], acc.at[dstx], add=True)

    plsc.subcore_barrier()

    pltpu.sync_copy(acc.at[pl.ds(s * DZ, DZ)], zb.at[pl.ds(0, DZ)])
    pltpu.sync_copy(zb.at[pl.ds(0, DZ)], out_hbm.at[pl.ds(c * N + s * DZ, DZ)])

    @pl.when(s == NS - 1)
    def _():
        pltpu.sync_copy(acc.at[pl.ds(DZ * NS, 16)], zb.at[pl.ds(0, 16)])
        pltpu.sync_copy(zb.at[pl.ds(0, 16)],
                        out_hbm.at[pl.ds(c * N + DZ * NS, 16)])


NBP = 2504
IDXR = 88


def _make_sc_agg(mesh):
    return functools.partial(
        pl.kernel,
        out_type=jax.ShapeDtypeStruct((NC, N, D), jnp.float32),
        mesh=mesh,
        scratch_types=[
            pltpu.VMEM((IDXR, EB), jnp.int32),
            pltpu.VMEM((EB,), jnp.int32),
            pltpu.VMEM((EB,), jnp.int32),
            pltpu.VMEM((EB, D), jnp.float32),
            pltpu.VMEM((EB, D), jnp.float32),
            pltpu.VMEM_SHARED((N, D), jnp.float32),
            pltpu.SemaphoreType.DMA,
            pltpu.SemaphoreType.DMA,
        ],
    )(_sc_agg_body)


def _sc_agg_body(y_hbm, src2_hbm, dst_hbm, out_hbm, srcv, dstb0, dstb1,
                 rows0, rows1, acc, gsem0, gsem1):
    c = lax.axis_index("c")
    s = lax.axis_index("s")
    w = c * NS + s

    base = w * NBW
    abase = (base // 8) * 8
    off = base - abase

    pltpu.sync_copy(src2_hbm.at[pl.ds(abase, IDXR)], srcv)
    pltpu.sync_copy(dst_hbm.at[pl.ds(base * EB, EB)], dstb0)
    pltpu.async_copy(y_hbm.at[srcv.at[off]], rows1, gsem1)

    def zrow(i, carry):
        def zcol(j, carry2):
            rows0[i, pl.ds(j * 16, 16)] = jnp.zeros((16,), jnp.float32)
            return carry2

        lax.fori_loop(0, D // 16, zcol, 0)
        return carry

    lax.fori_loop(0, EB, zrow, 0)

    for k in range(NCH):
        pltpu.sync_copy(rows0.at[pl.ds(0, ACH)],
                        acc.at[pl.ds(s * ARS + k * ACH, ACH)])

    @pl.when(s == NS - 1)
    def _():
        pltpu.sync_copy(rows0.at[pl.ds(0, TAIL)], acc.at[pl.ds(ARS * NS, TAIL)])

    plsc.subcore_barrier()

    def pair(k, carry):
        g0 = 2 * k
        pltpu.make_async_copy(y_hbm.at[srcv.at[off + g0]], rows1,
                              gsem1).wait()
        pltpu.async_copy(y_hbm.at[srcv.at[off + g0 + 1]], rows0, gsem0)
        pltpu.sync_copy(dst_hbm.at[pl.ds((base + g0 + 1) * EB, EB)], dstb1)
        pltpu.sync_copy(rows1, acc.at[dstb0], add=True)
        pltpu.make_async_copy(y_hbm.at[srcv.at[off + g0 + 1]], rows0,
                              gsem0).wait()

        @pl.when(g0 + 2 < NBW)
        def _():
            pltpu.async_copy(y_hbm.at[srcv.at[off + g0 + 2]], rows1, gsem1)
            pltpu.sync_copy(dst_hbm.at[pl.ds((base + g0 + 2) * EB, EB)], dstb0)

        pltpu.sync_copy(rows0, acc.at[dstb1], add=True)
        return carry

    lax.fori_loop(0, NBW // 2, pair, 0)

    @pl.when(w == NW - 1)
    def _():
        def extra(i, carry):
            pltpu.sync_copy(dst_hbm.at[pl.ds((NBW * NW) * EB + i * EB, EB)],
                            dstb0)
            pltpu.async_copy(y_hbm.at[srcv.at[NBW * NW - abase + i]], rows0,
                             gsem0).wait()
            pltpu.sync_copy(rows0, acc.at[dstb0], add=True)
            return carry

        lax.fori_loop(0, NBX, extra, 0)

    plsc.subcore_barrier()

    for k in range(NCH):
        r0 = s * ARS + k * ACH
        pltpu.sync_copy(acc.at[pl.ds(r0, ACH)], rows0.at[pl.ds(0, ACH)])
        pltpu.sync_copy(rows0.at[pl.ds(0, ACH)], out_hbm.at[c, pl.ds(r0, ACH)])

    @pl.when(s == NS - 1)
    def _():
        r0 = ARS * NS
        pltpu.sync_copy(acc.at[pl.ds(r0, TAIL)], rows1.at[pl.ds(0, TAIL)])
        pltpu.sync_copy(rows1.at[pl.ds(0, TAIL)], out_hbm.at[c, pl.ds(r0, TAIL)])


def _tc1_body(degp_ref, x_ref, w_ref, y_ref, dinv_ref):
    deg = degp_ref[0] + degp_ref[1] + 1.0
    dinv = lax.rsqrt(jnp.maximum(deg, 1e-12))
    xw = jnp.dot(x_ref[...], w_ref[...], preferred_element_type=jnp.float32)
    y_ref[...] = xw * dinv
    dinv_ref[...] = dinv


def _tc1(degp, x, W1):
    return pl.pallas_call(
        _tc1_body,
        grid=(G,),
        in_specs=[
            pl.BlockSpec((NC, RB, 1), lambda i: (0, i, 0)),
            pl.BlockSpec((RB, D), lambda i: (i, 0)),
            pl.BlockSpec((D, D), lambda i: (0, 0)),
        ],
        out_specs=[
            pl.BlockSpec((RB, D), lambda i: (i, 0)),
            pl.BlockSpec((RB, 1), lambda i: (i, 0)),
        ],
        out_shape=[
            jax.ShapeDtypeStruct((N, D), jnp.float32),
            jax.ShapeDtypeStruct((N, 1), jnp.float32),
        ],
    )(degp, x, W1)


def _tc_mid_body(a_ref, y_ref, dinv_ref, b_ref, w_ref, yn_ref):
    dinv = dinv_ref[...]
    h = dinv * (a_ref[0] + a_ref[1] + y_ref[...]) + b_ref[...]
    h = jnp.maximum(h, 0.0)
    yn_ref[...] = jnp.dot(h, w_ref[...],
                          preferred_element_type=jnp.float32) * dinv


def _tc_mid(a, y, dinv, b, W):
    return pl.pallas_call(
        _tc_mid_body,
        grid=(G,),
        in_specs=[
            pl.BlockSpec((NC, RB, D), lambda i: (0, i, 0)),
            pl.BlockSpec((RB, D), lambda i: (i, 0)),
            pl.BlockSpec((RB, 1), lambda i: (i, 0)),
            pl.BlockSpec((1, D), lambda i: (0, 0)),
            pl.BlockSpec((D, D), lambda i: (0, 0)),
        ],
        out_specs=pl.BlockSpec((RB, D), lambda i: (i, 0)),
        out_shape=jax.ShapeDtypeStruct((N, D), jnp.float32),
    )(a, y, dinv, b, W)


def _tc_final_body(a_ref, y_ref, dinv_ref, b3_ref, wf1_ref, bf1_ref, wf2_ref,
                   bf2_ref, out_ref):
    h = dinv_ref[...] * (a_ref[0] + a_ref[1] + y_ref[...]) + b3_ref[...]
    h = jnp.maximum(h, 0.0)
    z = jnp.dot(h, wf1_ref[...],
                preferred_element_type=jnp.float32) + bf1_ref[...]
    z = jnp.maximum(z, 0.0)
    out_ref[...] = jnp.dot(z, wf2_ref[...],
                           preferred_element_type=jnp.float32) + bf2_ref[...]


def _tc_final(a, y, dinv, b3, Wf1, bf1, Wf2, bf2):
    return pl.pallas_call(
        _tc_final_body,
        grid=(G,),
        in_specs=[
            pl.BlockSpec((NC, RB, D), lambda i: (0, i, 0)),
            pl.BlockSpec((RB, D), lambda i: (i, 0)),
            pl.BlockSpec((RB, 1), lambda i: (i, 0)),
            pl.BlockSpec((1, D), lambda i: (0, 0)),
            pl.BlockSpec((D, D), lambda i: (0, 0)),
            pl.BlockSpec((1, D), lambda i: (0, 0)),
            pl.BlockSpec((D, NCLS), lambda i: (0, 0)),
            pl.BlockSpec((1, NCLS), lambda i: (0, 0)),
        ],
        out_specs=pl.BlockSpec((RB, NCLS), lambda i: (i, 0)),
        out_shape=jax.ShapeDtypeStruct((N, NCLS), jnp.float32),
    )(a, y, dinv, b3, Wf1, bf1, Wf2, bf2)


def kernel(x, edge_index, batch, W1, b1, W2, b2, W3, b3, Wf1, bf1, Wf2, bf2):
    src = edge_index[0]
    dst = edge_index[1]

    src2 = jnp.pad(src.reshape(NB, EB), ((0, NBP - NB), (0, 0)))

    _sc_deg, _sc_agg = _sc_kernels()
    degp = _sc_deg(dst)
    y1, dinv = _tc1(degp.reshape(NC, N, 1), x, W1)
    a1 = _sc_agg(y1, src2, dst)
    y2 = _tc_mid(a1, y1, dinv, b1.reshape(1, D), W2)
    a2 = _sc_agg(y2, src2, dst)
    y3 = _tc_mid(a2, y2, dinv, b2.reshape(1, D), W3)
    a3 = _sc_agg(y3, src2, dst)
    out = _tc_final(a3, y3, dinv, b3.reshape(1, D), Wf1, bf1.reshape(1, D),
                    Wf2, bf2.reshape(1, NCLS))
    return out


# speedup vs baseline: 1.0670x; 1.0093x over previous
import functools

import jax
import jax.numpy as jnp
from jax import lax
from jax.experimental import pallas as pl
from jax.experimental.pallas import tpu as pltpu
from jax.experimental.pallas import tpu_sc as plsc

N = 10000
E = 320000
D = 128
NCLS = 40

NC = 2
NS = 16
NW = NC * NS
EPW = E // NW
EB = 128
NFULL = EPW // EB
EREM = EPW - NFULL * EB
NB = E // EB
NBW = NB // NW
NBX = NB - NBW * NW
ARS = 624
ACH = 104
NCH = ARS // ACH
TAIL = N - ARS * NS
DZ = 624

RB = 2000
G = N // RB

@functools.lru_cache(maxsize=None)
def _sc_kernels():
    mesh = plsc.VectorSubcoreMesh(core_axis_name="c", subcore_axis_name="s",
                                  num_cores=NC, num_subcores=NS)
    deg = _make_sc_deg(mesh)
    agg = _make_sc_agg(mesh)
    return deg, agg


def _make_sc_deg(mesh):
    return functools.partial(
        pl.kernel,
        out_type=jax.ShapeDtypeStruct((NC * N,), jnp.float32),
        mesh=mesh,
        scratch_types=[
            pltpu.VMEM((NBW * EB,), jnp.int32),
            pltpu.VMEM((NBX * EB,), jnp.int32),
            pltpu.VMEM((NBW * EB,), jnp.float32),
            pltpu.VMEM((DZ + 16,), jnp.float32),
            pltpu.VMEM_SHARED((N,), jnp.float32),
            pltpu.SemaphoreType.DMA,
        ],
    )(_sc_deg_body)


def _sc_deg_body(dst_hbm, out_hbm, dstv, dstx, ones, zb, acc, sem):
    c = lax.axis_index("c")
    s = lax.axis_index("s")
    w = c * NS + s

    base = w * NBW * EB
    pltpu.async_copy(dst_hbm.at[pl.ds(base, NBW * EB)], dstv, sem)

    @pl.when(w == NW - 1)
    def _():
        pltpu.sync_copy(dst_hbm.at[pl.ds(NBW * NW * EB, NBX * EB)], dstx)

    def fill_z(i, carry):
        zb[pl.ds(i * 16, 16)] = jnp.zeros((16,), jnp.float32)
        return carry

    lax.fori_loop(0, (DZ + 16) // 16, fill_z, 0)

    def fill_o(i, carry):
        ones[pl.ds(i * 16, 16)] = jnp.ones((16,), jnp.float32)
        return carry

    lax.fori_loop(0, NBW * EB // 16, fill_o, 0)

    pltpu.sync_copy(zb.at[pl.ds(0, DZ)], acc.at[pl.ds(s * DZ, DZ)])

    @pl.when(s == NS - 1)
    def _():
        pltpu.sync_copy(zb.at[pl.ds(0, 16)], acc.at[pl.ds(DZ * NS, 16)])

    plsc.subcore_barrier()
    pltpu.make_async_copy(dst_hbm.at[pl.ds(base, NBW * EB)], dstv, sem).wait()

    pltpu.sync_copy(ones, acc.at[dstv], add=True)

    @pl.when(w == NW - 1)
    def _():
        pltpu.sync_copy(ones.at[pl.ds(0, NBX * EB)], acc.at[dstx], add=True)

    plsc.subcore_barrier()

    pltpu.sync_copy(acc.at[pl.ds(s * DZ, DZ)], zb.at[pl.ds(0, DZ)])
    pltpu.sync_copy(zb.at[pl.ds(0, DZ)], out_hbm.at[pl.ds(c * N + s * DZ, DZ)])

    @pl.when(s == NS - 1)
    def _():
        pltpu.sync_copy(acc.at[pl.ds(DZ * NS, 16)], zb.at[pl.ds(0, 16)])
        pltpu.sync_copy(zb.at[pl.ds(0, 16)],
                        out_hbm.at[pl.ds(c * N + DZ * NS, 16)])


NBP = 2504
IDXR = 88


def _make_sc_agg(mesh):
    return functools.partial(
        pl.kernel,
        out_type=jax.ShapeDtypeStruct((NC, N, D), jnp.float32),
        mesh=mesh,
        scratch_types=[
            pltpu.VMEM((IDXR, EB), jnp.int32),
            pltpu.VMEM((EB,), jnp.int32),
            pltpu.VMEM((EB,), jnp.int32),
            pltpu.VMEM((EB, D), jnp.float32),
            pltpu.VMEM((EB, D), jnp.float32),
            pltpu.VMEM_SHARED((N, D), jnp.float32),
            pltpu.SemaphoreType.DMA,
            pltpu.SemaphoreType.DMA,
            pltpu.SemaphoreType.DMA,
            pltpu.SemaphoreType.DMA,
        ],
    )(_sc_agg_body)


def _sc_agg_body(y_hbm, src2_hbm, dst_hbm, out_hbm, srcv, dstb0, dstb1,
                 rows0, rows1, acc, gsem0, gsem1, dsem0, dsem1):
    c = lax.axis_index("c")
    s = lax.axis_index("s")
    w = c * NS + s

    base = w * NBW
    abase = (base // 8) * 8
    off = base - abase

    pltpu.sync_copy(src2_hbm.at[pl.ds(abase, IDXR)], srcv)
    pltpu.async_copy(dst_hbm.at[pl.ds(base * EB, EB)], dstb0, dsem0)
    pltpu.async_copy(y_hbm.at[srcv.at[off]], rows1, gsem1)

    def zrow(i, carry):
        def zcol(j, carry2):
            rows0[i, pl.ds(j * 16, 16)] = jnp.zeros((16,), jnp.float32)
            return carry2

        lax.fori_loop(0, D // 16, zcol, 0)
        return carry

    lax.fori_loop(0, EB, zrow, 0)

    for k in range(NCH):
        pltpu.sync_copy(rows0.at[pl.ds(0, ACH)],
                        acc.at[pl.ds(s * ARS + k * ACH, ACH)])

    @pl.when(s == NS - 1)
    def _():
        pltpu.sync_copy(rows0.at[pl.ds(0, TAIL)], acc.at[pl.ds(ARS * NS, TAIL)])

    plsc.subcore_barrier()

    def pair(k, carry):
        g0 = 2 * k
        pltpu.make_async_copy(y_hbm.at[srcv.at[off + g0]], rows1,
                              gsem1).wait()
        pltpu.async_copy(y_hbm.at[srcv.at[off + g0 + 1]], rows0, gsem0)
        pltpu.async_copy(dst_hbm.at[pl.ds((base + g0 + 1) * EB, EB)], dstb1,
                         dsem1)
        pltpu.make_async_copy(dst_hbm.at[pl.ds((base + g0) * EB, EB)], dstb0,
                              dsem0).wait()
        pltpu.sync_copy(rows1, acc.at[dstb0], add=True)
        pltpu.make_async_copy(y_hbm.at[srcv.at[off + g0 + 1]], rows0,
                              gsem0).wait()

        @pl.when(g0 + 2 < NBW)
        def _():
            pltpu.async_copy(y_hbm.at[srcv.at[off + g0 + 2]], rows1, gsem1)
            pltpu.async_copy(dst_hbm.at[pl.ds((base + g0 + 2) * EB, EB)],
                             dstb0, dsem0)

        pltpu.make_async_copy(dst_hbm.at[pl.ds((base + g0 + 1) * EB, EB)],
                              dstb1, dsem1).wait()
        pltpu.sync_copy(rows0, acc.at[dstb1], add=True)
        return carry

    lax.fori_loop(0, NBW // 2, pair, 0)

    @pl.when(w == NW - 1)
    def _():
        def extra(i, carry):
            pltpu.sync_copy(dst_hbm.at[pl.ds((NBW * NW) * EB + i * EB, EB)],
                            dstb0)
            pltpu.async_copy(y_hbm.at[srcv.at[NBW * NW - abase + i]], rows0,
                             gsem0).wait()
            pltpu.sync_copy(rows0, acc.at[dstb0], add=True)
            return carry

        lax.fori_loop(0, NBX, extra, 0)

    plsc.subcore_barrier()

    for k in range(NCH):
        r0 = s * ARS + k * ACH
        pltpu.sync_copy(acc.at[pl.ds(r0, ACH)], rows0.at[pl.ds(0, ACH)])
        pltpu.sync_copy(rows0.at[pl.ds(0, ACH)], out_hbm.at[c, pl.ds(r0, ACH)])

    @pl.when(s == NS - 1)
    def _():
        r0 = ARS * NS
        pltpu.sync_copy(acc.at[pl.ds(r0, TAIL)], rows1.at[pl.ds(0, TAIL)])
        pltpu.sync_copy(rows1.at[pl.ds(0, TAIL)], out_hbm.at[c, pl.ds(r0, TAIL)])


def _tc1_body(degp_ref, x_ref, w_ref, y_ref, dinv_ref):
    deg = degp_ref[0] + degp_ref[1] + 1.0
    dinv = lax.rsqrt(jnp.maximum(deg, 1e-12))
    xw = jnp.dot(x_ref[...], w_ref[...], preferred_element_type=jnp.float32)
    y_ref[...] = xw * dinv
    dinv_ref[...] = dinv


def _tc1(degp, x, W1):
    return pl.pallas_call(
        _tc1_body,
        grid=(G,),
        in_specs=[
            pl.BlockSpec((NC, RB, 1), lambda i: (0, i, 0)),
            pl.BlockSpec((RB, D), lambda i: (i, 0)),
            pl.BlockSpec((D, D), lambda i: (0, 0)),
        ],
        out_specs=[
            pl.BlockSpec((RB, D), lambda i: (i, 0)),
            pl.BlockSpec((RB, 1), lambda i: (i, 0)),
        ],
        out_shape=[
            jax.ShapeDtypeStruct((N, D), jnp.float32),
            jax.ShapeDtypeStruct((N, 1), jnp.float32),
        ],
    )(degp, x, W1)


def _tc_mid_body(a_ref, y_ref, dinv_ref, b_ref, w_ref, yn_ref):
    dinv = dinv_ref[...]
    h = dinv * (a_ref[0] + a_ref[1] + y_ref[...]) + b_ref[...]
    h = jnp.maximum(h, 0.0)
    yn_ref[...] = jnp.dot(h, w_ref[...],
                          preferred_element_type=jnp.float32) * dinv


def _tc_mid(a, y, dinv, b, W):
    return pl.pallas_call(
        _tc_mid_body,
        grid=(G,),
        in_specs=[
            pl.BlockSpec((NC, RB, D), lambda i: (0, i, 0)),
            pl.BlockSpec((RB, D), lambda i: (i, 0)),
            pl.BlockSpec((RB, 1), lambda i: (i, 0)),
            pl.BlockSpec((1, D), lambda i: (0, 0)),
            pl.BlockSpec((D, D), lambda i: (0, 0)),
        ],
        out_specs=pl.BlockSpec((RB, D), lambda i: (i, 0)),
        out_shape=jax.ShapeDtypeStruct((N, D), jnp.float32),
    )(a, y, dinv, b, W)


def _tc_final_body(a_ref, y_ref, dinv_ref, b3_ref, wf1_ref, bf1_ref, wf2_ref,
                   bf2_ref, out_ref):
    h = dinv_ref[...] * (a_ref[0] + a_ref[1] + y_ref[...]) + b3_ref[...]
    h = jnp.maximum(h, 0.0)
    z = jnp.dot(h, wf1_ref[...],
                preferred_element_type=jnp.float32) + bf1_ref[...]
    z = jnp.maximum(z, 0.0)
    out_ref[...] = jnp.dot(z, wf2_ref[...],
                           preferred_element_type=jnp.float32) + bf2_ref[...]


def _tc_final(a, y, dinv, b3, Wf1, bf1, Wf2, bf2):
    return pl.pallas_call(
        _tc_final_body,
        grid=(G,),
        in_specs=[
            pl.BlockSpec((NC, RB, D), lambda i: (0, i, 0)),
            pl.BlockSpec((RB, D), lambda i: (i, 0)),
            pl.BlockSpec((RB, 1), lambda i: (i, 0)),
            pl.BlockSpec((1, D), lambda i: (0, 0)),
            pl.BlockSpec((D, D), lambda i: (0, 0)),
            pl.BlockSpec((1, D), lambda i: (0, 0)),
            pl.BlockSpec((D, NCLS), lambda i: (0, 0)),
            pl.BlockSpec((1, NCLS), lambda i: (0, 0)),
        ],
        out_specs=pl.BlockSpec((RB, NCLS), lambda i: (i, 0)),
        out_shape=jax.ShapeDtypeStruct((N, NCLS), jnp.float32),
    )(a, y, dinv, b3, Wf1, bf1, Wf2, bf2)


def kernel(x, edge_index, batch, W1, b1, W2, b2, W3, b3, Wf1, bf1, Wf2, bf2):
    src = edge_index[0]
    dst = edge_index[1]

    src2 = jnp.pad(src.reshape(NB, EB), ((0, NBP - NB), (0, 0)))

    _sc_deg, _sc_agg = _sc_kernels()
    degp = _sc_deg(dst)
    y1, dinv = _tc1(degp.reshape(NC, N, 1), x, W1)
    a1 = _sc_agg(y1, src2, dst)
    y2 = _tc_mid(a1, y1, dinv, b1.reshape(1, D), W2)
    a2 = _sc_agg(y2, src2, dst)
    y3 = _tc_mid(a2, y2, dinv, b2.reshape(1, D), W3)
    a3 = _sc_agg(y3, src2, dst)
    out = _tc_final(a3, y3, dinv, b3.reshape(1, D), Wf1, bf1.reshape(1, D),
                    Wf2, bf2.reshape(1, NCLS))
    return out


# speedup vs baseline: 1.1364x; 1.0650x over previous
import functools

import jax
import jax.numpy as jnp
from jax import lax
from jax.experimental import pallas as pl
from jax.experimental.pallas import tpu as pltpu
from jax.experimental.pallas import tpu_sc as plsc

N = 10000
E = 320000
D = 128
NCLS = 40

NC = 2
NS = 16
NW = NC * NS
EPW = E // NW
EB = 128
NFULL = EPW // EB
EREM = EPW - NFULL * EB
NB = E // EB
NBW = NB // NW
NBX = NB - NBW * NW
ARS = 624
ACH = 104
NCH = ARS // ACH
TAIL = N - ARS * NS
DZ = 624

RB = 2000
G = N // RB

@functools.lru_cache(maxsize=None)
def _sc_kernels():
    mesh = plsc.VectorSubcoreMesh(core_axis_name="c", subcore_axis_name="s",
                                  num_cores=NC, num_subcores=NS)
    deg = _make_sc_deg(mesh)
    agg = _make_sc_agg(mesh)
    return deg, agg


def _make_sc_deg(mesh):
    return functools.partial(
        pl.kernel,
        out_type=jax.ShapeDtypeStruct((NC * N,), jnp.float32),
        mesh=mesh,
        scratch_types=[
            pltpu.VMEM((NBW * EB,), jnp.int32),
            pltpu.VMEM((NBX * EB,), jnp.int32),
            pltpu.VMEM((NBW * EB,), jnp.float32),
            pltpu.VMEM((DZ + 16,), jnp.float32),
            pltpu.VMEM_SHARED((N,), jnp.float32),
            pltpu.SemaphoreType.DMA,
        ],
    )(_sc_deg_body)


def _sc_deg_body(dst_hbm, out_hbm, dstv, dstx, ones, zb, acc, sem):
    c = lax.axis_index("c")
    s = lax.axis_index("s")
    w = c * NS + s

    base = w * NBW * EB
    pltpu.async_copy(dst_hbm.at[pl.ds(base, NBW * EB)], dstv, sem)

    @pl.when(w == NW - 1)
    def _():
        pltpu.sync_copy(dst_hbm.at[pl.ds(NBW * NW * EB, NBX * EB)], dstx)

    def fill_z(i, carry):
        zb[pl.ds(i * 16, 16)] = jnp.zeros((16,), jnp.float32)
        return carry

    lax.fori_loop(0, (DZ + 16) // 16, fill_z, 0)

    def fill_o(i, carry):
        ones[pl.ds(i * 16, 16)] = jnp.ones((16,), jnp.float32)
        return carry

    lax.fori_loop(0, NBW * EB // 16, fill_o, 0)

    pltpu.sync_copy(zb.at[pl.ds(0, DZ)], acc.at[pl.ds(s * DZ, DZ)])

    @pl.when(s == NS - 1)
    def _():
        pltpu.sync_copy(zb.at[pl.ds(0, 16)], acc.at[pl.ds(DZ * NS, 16)])

    plsc.subcore_barrier()
    pltpu.make_async_copy(dst_hbm.at[pl.ds(base, NBW * EB)], dstv, sem).wait()

    pltpu.sync_copy(ones, acc.at[dstv], add=True)

    @pl.when(w == NW - 1)
    def _():
        pltpu.sync_copy(ones.at[pl.ds(0, NBX * EB)], acc.at[dstx], add=True)

    plsc.subcore_barrier()

    pltpu.sync_copy(acc.at[pl.ds(s * DZ, DZ)], zb.at[pl.ds(0, DZ)])
    pltpu.sync_copy(zb.at[pl.ds(0, DZ)], out_hbm.at[pl.ds(c * N + s * DZ, DZ)])

    @pl.when(s == NS - 1)
    def _():
        pltpu.sync_copy(acc.at[pl.ds(DZ * NS, 16)], zb.at[pl.ds(0, 16)])
        pltpu.sync_copy(zb.at[pl.ds(0, 16)],
                        out_hbm.at[pl.ds(c * N + DZ * NS, 16)])


NBP = 2504
IDXR = 88


def _make_sc_agg(mesh):
    return functools.partial(
        pl.kernel,
        out_type=jax.ShapeDtypeStruct((NC, N, D), jnp.float32),
        mesh=mesh,
        scratch_types=[
            [pltpu.VMEM((EB,), jnp.int32) for _ in range(3)],
            [pltpu.VMEM((EB,), jnp.int32) for _ in range(3)],
            [pltpu.VMEM((EB, D), jnp.float32) for _ in range(3)],
            pltpu.VMEM_SHARED((N, D), jnp.float32),
            [pltpu.SemaphoreType.DMA for _ in range(3)],
            [pltpu.SemaphoreType.DMA for _ in range(3)],
            [pltpu.SemaphoreType.DMA for _ in range(3)],
        ],
    )(_sc_agg_body)


def _sc_agg_body(y_hbm, src_hbm, dst_hbm, out_hbm, srcb, dstb, rows, acc,
                 isem, dsem, gsem):
    c = lax.axis_index("c")
    s = lax.axis_index("s")
    w = c * NS + s

    base = w * NBW

    def load_idx(g, r):
        pltpu.async_copy(src_hbm.at[pl.ds((base + g) * EB, EB)], srcb[r],
                         isem[r])
        pltpu.async_copy(dst_hbm.at[pl.ds((base + g) * EB, EB)], dstb[r],
                         dsem[r])

    def start_gather(g, r):
        pltpu.make_async_copy(src_hbm.at[pl.ds((base + g) * EB, EB)], srcb[r],
                              isem[r]).wait()
        pltpu.async_copy(y_hbm.at[srcb[r]], rows[r], gsem[r])

    def finish_batch(g, r):
        pltpu.make_async_copy(y_hbm.at[srcb[r]], rows[r], gsem[r]).wait()
        pltpu.make_async_copy(dst_hbm.at[pl.ds((base + g) * EB, EB)], dstb[r],
                              dsem[r]).wait()
        pltpu.sync_copy(rows[r], acc.at[dstb[r]], add=True)

    for r in range(3):
        load_idx(r, r)
    start_gather(0, 0)
    start_gather(1, 1)

    def zrow(i, carry):
        def zcol(j, carry2):
            rows[2][i, pl.ds(j * 16, 16)] = jnp.zeros((16,), jnp.float32)
            return carry2

        lax.fori_loop(0, D // 16, zcol, 0)
        return carry

    lax.fori_loop(0, EB, zrow, 0)

    for k in range(NCH):
        pltpu.sync_copy(rows[2].at[pl.ds(0, ACH)],
                        acc.at[pl.ds(s * ARS + k * ACH, ACH)])

    @pl.when(s == NS - 1)
    def _():
        pltpu.sync_copy(rows[2].at[pl.ds(0, TAIL)],
                        acc.at[pl.ds(ARS * NS, TAIL)])

    plsc.subcore_barrier()

    def triple(k, carry):
        g0 = 3 * k
        for j in range(3):
            r = j
            r2 = (j + 2) % 3

            @pl.when(g0 + j + 2 < NBW)
            def _():
                start_gather(g0 + j + 2, r2)

            finish_batch(g0 + j, r)

            @pl.when(g0 + j + 3 < NBW)
            def _():
                load_idx(g0 + j + 3, r)

        return carry

    lax.fori_loop(0, NBW // 3, triple, 0)

    @pl.when(w == NW - 1)
    def _():
        def extra(i, carry):
            e0 = (NBW * NW + i) * EB
            pltpu.sync_copy(src_hbm.at[pl.ds(e0, EB)], srcb[0])
            pltpu.sync_copy(dst_hbm.at[pl.ds(e0, EB)], dstb[0])
            pltpu.async_copy(y_hbm.at[srcb[0]], rows[0], gsem[0]).wait()
            pltpu.sync_copy(rows[0], acc.at[dstb[0]], add=True)
            return carry

        lax.fori_loop(0, NBX, extra, 0)

    plsc.subcore_barrier()

    for k in range(NCH):
        r0 = s * ARS + k * ACH
        pltpu.sync_copy(acc.at[pl.ds(r0, ACH)], rows[0].at[pl.ds(0, ACH)])
        pltpu.sync_copy(rows[0].at[pl.ds(0, ACH)],
                        out_hbm.at[c, pl.ds(r0, ACH)])

    @pl.when(s == NS - 1)
    def _():
        r0 = ARS * NS
        pltpu.sync_copy(acc.at[pl.ds(r0, TAIL)], rows[1].at[pl.ds(0, TAIL)])
        pltpu.sync_copy(rows[1].at[pl.ds(0, TAIL)],
                        out_hbm.at[c, pl.ds(r0, TAIL)])


def _tc1_body(degp_ref, x_ref, w_ref, y_ref, dinv_ref):
    deg = degp_ref[0] + degp_ref[1] + 1.0
    dinv = lax.rsqrt(jnp.maximum(deg, 1e-12))
    xw = jnp.dot(x_ref[...], w_ref[...], preferred_element_type=jnp.float32)
    y_ref[...] = xw * dinv
    dinv_ref[...] = dinv


def _tc1(degp, x, W1):
    return pl.pallas_call(
        _tc1_body,
        grid=(G,),
        in_specs=[
            pl.BlockSpec((NC, RB, 1), lambda i: (0, i, 0)),
            pl.BlockSpec((RB, D), lambda i: (i, 0)),
            pl.BlockSpec((D, D), lambda i: (0, 0)),
        ],
        out_specs=[
            pl.BlockSpec((RB, D), lambda i: (i, 0)),
            pl.BlockSpec((RB, 1), lambda i: (i, 0)),
        ],
        out_shape=[
            jax.ShapeDtypeStruct((N, D), jnp.float32),
            jax.ShapeDtypeStruct((N, 1), jnp.float32),
        ],
    )(degp, x, W1)


def _tc_mid_body(a_ref, y_ref, dinv_ref, b_ref, w_ref, yn_ref):
    dinv = dinv_ref[...]
    h = dinv * (a_ref[0] + a_ref[1] + y_ref[...]) + b_ref[...]
    h = jnp.maximum(h, 0.0)
    yn_ref[...] = jnp.dot(h, w_ref[...],
                          preferred_element_type=jnp.float32) * dinv


def _tc_mid(a, y, dinv, b, W):
    return pl.pallas_call(
        _tc_mid_body,
        grid=(G,),
        in_specs=[
            pl.BlockSpec((NC, RB, D), lambda i: (0, i, 0)),
            pl.BlockSpec((RB, D), lambda i: (i, 0)),
            pl.BlockSpec((RB, 1), lambda i: (i, 0)),
            pl.BlockSpec((1, D), lambda i: (0, 0)),
            pl.BlockSpec((D, D), lambda i: (0, 0)),
        ],
        out_specs=pl.BlockSpec((RB, D), lambda i: (i, 0)),
        out_shape=jax.ShapeDtypeStruct((N, D), jnp.float32),
    )(a, y, dinv, b, W)


def _tc_final_body(a_ref, y_ref, dinv_ref, b3_ref, wf1_ref, bf1_ref, wf2_ref,
                   bf2_ref, out_ref):
    h = dinv_ref[...] * (a_ref[0] + a_ref[1] + y_ref[...]) + b3_ref[...]
    h = jnp.maximum(h, 0.0)
    z = jnp.dot(h, wf1_ref[...],
                preferred_element_type=jnp.float32) + bf1_ref[...]
    z = jnp.maximum(z, 0.0)
    out_ref[...] = jnp.dot(z, wf2_ref[...],
                           preferred_element_type=jnp.float32) + bf2_ref[...]


def _tc_final(a, y, dinv, b3, Wf1, bf1, Wf2, bf2):
    return pl.pallas_call(
        _tc_final_body,
        grid=(G,),
        in_specs=[
            pl.BlockSpec((NC, RB, D), lambda i: (0, i, 0)),
            pl.BlockSpec((RB, D), lambda i: (i, 0)),
            pl.BlockSpec((RB, 1), lambda i: (i, 0)),
            pl.BlockSpec((1, D), lambda i: (0, 0)),
            pl.BlockSpec((D, D), lambda i: (0, 0)),
            pl.BlockSpec((1, D), lambda i: (0, 0)),
            pl.BlockSpec((D, NCLS), lambda i: (0, 0)),
            pl.BlockSpec((1, NCLS), lambda i: (0, 0)),
        ],
        out_specs=pl.BlockSpec((RB, NCLS), lambda i: (i, 0)),
        out_shape=jax.ShapeDtypeStruct((N, NCLS), jnp.float32),
    )(a, y, dinv, b3, Wf1, bf1, Wf2, bf2)


def kernel(x, edge_index, batch, W1, b1, W2, b2, W3, b3, Wf1, bf1, Wf2, bf2):
    src = edge_index[0]
    dst = edge_index[1]

    _sc_deg, _sc_agg = _sc_kernels()
    degp = _sc_deg(dst)
    y1, dinv = _tc1(degp.reshape(NC, N, 1), x, W1)
    a1 = _sc_agg(y1, src, dst)
    y2 = _tc_mid(a1, y1, dinv, b1.reshape(1, D), W2)
    a2 = _sc_agg(y2, src, dst)
    y3 = _tc_mid(a2, y2, dinv, b2.reshape(1, D), W3)
    a3 = _sc_agg(y3, src, dst)
    out = _tc_final(a3, y3, dinv, b3.reshape(1, D), Wf1, bf1.reshape(1, D),
                    Wf2, bf2.reshape(1, NCLS))
    return out
